# baseline (speedup 1.0000x reference)
import functools

import jax
import jax.numpy as jnp
from jax import lax
from jax.experimental import pallas as pl
from jax.experimental.pallas import tpu as pltpu
from jax.experimental.pallas import tpu_sc as plsc

RANK = 16
LANES = 16
IDX_CHUNK = 128


@functools.lru_cache(maxsize=None)
def _make_kernel(batch: int):
    info = plsc.get_sparse_core_info()
    num_cores, num_subcores = info.num_cores, info.num_subcores
    nw = num_cores * num_subcores
    assert batch % (8 * nw) == 0
    bpw = batch // nw
    nch = bpw // IDX_CHUNK

    mesh = plsc.VectorSubcoreMesh(core_axis_name="c", subcore_axis_name="s")

    @functools.partial(
        pl.kernel,
        mesh=mesh,
        out_type=jax.ShapeDtypeStruct((batch,), jnp.float32),
        compiler_params=pltpu.CompilerParams(
            needs_layout_passes=False, use_tc_tiling_on_sc=False
        ),
        scratch_types=[
            pltpu.VMEM((bpw,), jnp.int32),
            pltpu.VMEM((bpw,), jnp.int32),
            pltpu.VMEM((RANK, bpw), jnp.float32),
            pltpu.VMEM((RANK, bpw), jnp.float32),
            pltpu.VMEM((bpw,), jnp.float32),
            pltpu.VMEM((bpw,), jnp.float32),
            pltpu.VMEM((bpw,), jnp.float32),
            pltpu.SemaphoreType.DMA,
        ],
    )
    def mf_kernel(user_hbm, item_hbm, uft_hbm, ift_hbm, ub_hbm, ib_hbm,
                  out_hbm, uidx, iidx, ufv, ifv, ubv, ibv, outv, sem):
        wid = lax.axis_index("s") * num_cores + lax.axis_index("c")
        base = wid * bpw
        pltpu.sync_copy(user_hbm.at[pl.ds(base, bpw)], uidx)
        pltpu.sync_copy(item_hbm.at[pl.ds(base, bpw)], iidx)

        copies = []
        for j in range(nch):
            sl = pl.ds(j * IDX_CHUNK, IDX_CHUNK)
            usl = uidx.at[sl]
            isl = iidx.at[sl]
            copies.append(pltpu.async_copy(ub_hbm.at[usl], ubv.at[sl], sem))
            copies.append(pltpu.async_copy(ib_hbm.at[isl], ibv.at[sl], sem))
            for c in range(RANK):
                copies.append(pltpu.async_copy(
                    uft_hbm.at[c].at[usl], ufv.at[c].at[sl], sem))
                copies.append(pltpu.async_copy(
                    ift_hbm.at[c].at[isl], ifv.at[c].at[sl], sem))
        for cp in copies:
            cp.wait()

        def step(s, carry):
            sl = pl.ds(s * LANES, LANES)
            acc = ubv[sl] + ibv[sl]
            for c in range(RANK):
                acc = acc + ufv[c, sl] * ifv[c, sl]
            outv[sl] = acc
            return carry

        lax.fori_loop(0, bpw // LANES, step, 0)
        pltpu.sync_copy(outv, out_hbm.at[pl.ds(base, bpw)])

    return mf_kernel


def kernel(user, item, user_factors, item_factors, user_biases, item_biases):
    batch = user.shape[0]
    k = _make_kernel(batch)
    return k(
        user.astype(jnp.int32),
        item.astype(jnp.int32),
        user_factors.T,
        item_factors.T,
        user_biases.reshape(-1),
        item_biases.reshape(-1),
    )


# speedup vs baseline: 2.8971x; 2.8971x over previous
import functools

import jax
import jax.numpy as jnp
from jax import lax
from jax.experimental import pallas as pl
from jax.experimental.pallas import tpu as pltpu
from jax.experimental.pallas import tpu_sc as plsc

RANK = 16
LANES = 16
IDX_CHUNK = 128


@functools.lru_cache(maxsize=None)
def _make_kernel(batch: int):
    info = plsc.get_sparse_core_info()
    num_cores, num_subcores = info.num_cores, info.num_subcores
    nw = num_cores * num_subcores
    assert batch % (8 * nw) == 0
    bpw = batch // nw
    nch = bpw // IDX_CHUNK

    mesh = plsc.VectorSubcoreMesh(core_axis_name="c", subcore_axis_name="s")

    @functools.partial(
        pl.kernel,
        mesh=mesh,
        out_type=jax.ShapeDtypeStruct((batch,), jnp.float32),
        compiler_params=pltpu.CompilerParams(
            needs_layout_passes=False, use_tc_tiling_on_sc=False
        ),
        scratch_types=[
            pltpu.VMEM((bpw,), jnp.int32),
            pltpu.VMEM((bpw,), jnp.int32),
            pltpu.VMEM((bpw, RANK), jnp.float32),
            pltpu.VMEM((bpw, RANK), jnp.float32),
            pltpu.VMEM((bpw,), jnp.float32),
            pltpu.VMEM((bpw,), jnp.float32),
            pltpu.VMEM((bpw,), jnp.float32),
            pltpu.SemaphoreType.DMA,
        ],
    )
    def mf_kernel(user_hbm, item_hbm, uf_hbm, if_hbm, ub_hbm, ib_hbm,
                  out_hbm, uidx, iidx, ufv, ifv, ubv, ibv, outv, sem):
        wid = lax.axis_index("s") * num_cores + lax.axis_index("c")
        base = wid * bpw
        pltpu.sync_copy(user_hbm.at[pl.ds(base, bpw)], uidx)
        pltpu.sync_copy(item_hbm.at[pl.ds(base, bpw)], iidx)

        copies = []
        for j in range(nch):
            sl = pl.ds(j * IDX_CHUNK, IDX_CHUNK)
            usl = uidx.at[sl]
            isl = iidx.at[sl]
            copies.append(pltpu.async_copy(uf_hbm.at[usl], ufv.at[sl], sem))
            copies.append(pltpu.async_copy(if_hbm.at[isl], ifv.at[sl], sem))
            copies.append(pltpu.async_copy(ub_hbm.at[usl], ubv.at[sl], sem))
            copies.append(pltpu.async_copy(ib_hbm.at[isl], ibv.at[sl], sem))
        for c in copies:
            c.wait()

        lane = lax.iota(jnp.int32, LANES)

        def step(s, carry):
            b0 = s * LANES
            acc = ubv[pl.ds(b0, LANES)] + ibv[pl.ds(b0, LANES)]
            for r in range(LANES):
                dot = jnp.sum(ufv[b0 + r] * ifv[b0 + r])
                acc = jnp.where(lane == r, acc + dot, acc)
            outv[pl.ds(b0, LANES)] = acc
            return carry

        lax.fori_loop(0, bpw // LANES, step, 0)
        pltpu.sync_copy(outv, out_hbm.at[pl.ds(base, bpw)])

    return mf_kernel


def kernel(user, item, user_factors, item_factors, user_biases, item_biases):
    batch = user.shape[0]
    k = _make_kernel(batch)
    return k(
        user.astype(jnp.int32),
        item.astype(jnp.int32),
        user_factors,
        item_factors,
        user_biases.reshape(-1),
        item_biases.reshape(-1),
    )
